# baseline (speedup 1.0000x reference)
import jax
import jax.numpy as jnp
from jax import lax
from jax.experimental import pallas as pl
from jax.experimental.pallas import tpu as pltpu
from jax.experimental.pallas import tpu_sc as plsc

NF = 26
VOCAB = 1000
B = 16384
D = 128
NW = 32
BPW = B // NW
G = 8
ROWS = G * NF
HALF = ROWS // 2
NGROUP = BPW // G
IDXW = BPW * NF


def _fm_body(cross_hbm, lin_hbm, idx_hbm, out_hbm,
             idx_v, lin_v, buf_a, buf_b, out_v, sem_a, sem_b):
    wid = lax.axis_index("s") * 2 + lax.axis_index("c")

    pltpu.sync_copy(idx_hbm.at[pl.ds(wid * IDXW, IDXW)],
                    idx_v.at[pl.ds(0, IDXW)])
    pltpu.sync_copy(lin_hbm, lin_v.at[pl.ds(0, NF * VOCAB)])

    def issue(gg, buf, sem):
        off = gg * ROWS
        pltpu.async_copy(cross_hbm.at[idx_v.at[pl.ds(off, ROWS)]],
                         buf, sem)

    def drain(buf, sem):
        pltpu.make_async_copy(cross_hbm.at[pl.ds(0, ROWS)], buf, sem).wait()

    lane = lax.iota(jnp.int32, 16)
    zero = jnp.zeros((16,), jnp.float32)

    def shuf(x, perm):
        return lax.gather(
            x, perm[:, None],
            lax.GatherDimensionNumbers(offset_dims=(),
                                       collapsed_slice_dims=(0,),
                                       start_index_map=(0,)),
            (1,), mode=lax.GatherScatterMode.PROMISE_IN_BOUNDS)

    def sample(buf, sbase, s):
        row0 = s * NF

        def frow(r, carry):
            accs = list(carry)
            for k in range(8):
                x = buf[row0 + r, pl.ds(k * 16, 16)]
                accs[k] = accs[k] + x
                accs[8 + k] = accs[8 + k] + x * x
            return tuple(accs)

        accs = lax.fori_loop(0, NF, frow, (zero,) * 16, unroll=2)
        t = zero
        for k in range(8):
            t = t + (accs[k] * accs[k] - accs[8 + k])

        soff = sbase + row0
        la = zero
        lb = zero
        for f in range(0, NF, 2):
            ja = idx_v[pl.ds(soff + f, 16)][0]
            jb = idx_v[pl.ds(soff + f + 1, 16)][0]
            la = la + lin_v[pl.ds(ja, 16)]
            lb = lb + lin_v[pl.ds(jb, 16)]
        u = t + 2.0 * jnp.where(lane == 0, la + lb, 0.0)
        for sh in (8, 4, 2, 1):
            u = u + shuf(u, lane ^ sh)

        return 0.5 * u

    def process(buf, r, lbase, sbase):
        def sbody(s, rr):
            rv = sample(buf, sbase, s)
            return jnp.where(lane == lbase + s, rv, rr)
        return lax.fori_loop(0, G, sbody, r)

    issue(0, buf_a, sem_a)

    def gbody(g2, c):
        gg0 = g2 * 2
        issue(gg0 + 1, buf_b, sem_b)
        drain(buf_a, sem_a)
        r = process(buf_a, zero, 0, gg0 * ROWS)

        @pl.when(g2 < NGROUP // 2 - 1)
        def _():
            issue(gg0 + 2, buf_a, sem_a)

        drain(buf_b, sem_b)
        r = process(buf_b, r, G, (gg0 + 1) * ROWS)
        out_v[pl.ds(g2 * 16, 16)] = r
        return c

    lax.fori_loop(0, NGROUP // 2, gbody, 0)

    pltpu.sync_copy(out_v, out_hbm.at[pl.ds(wid * BPW, BPW)])


def kernel(linear_tables, cross_tables, bias,
           feat_0, feat_1, feat_2, feat_3, feat_4, feat_5, feat_6,
           feat_7, feat_8, feat_9, feat_10, feat_11, feat_12, feat_13,
           feat_14, feat_15, feat_16, feat_17, feat_18, feat_19, feat_20,
           feat_21, feat_22, feat_23, feat_24, feat_25):
    feats = [feat_0, feat_1, feat_2, feat_3, feat_4, feat_5, feat_6,
             feat_7, feat_8, feat_9, feat_10, feat_11, feat_12, feat_13,
             feat_14, feat_15, feat_16, feat_17, feat_18, feat_19, feat_20,
             feat_21, feat_22, feat_23, feat_24, feat_25]
    idx = jnp.stack(feats, axis=1)
    flat_idx = (idx + (jnp.arange(NF, dtype=jnp.int32) * VOCAB)[None, :]
                ).reshape(-1)
    cross_flat = cross_tables.reshape(NF * VOCAB, D)
    lin_flat = linear_tables.reshape(NF * VOCAB)

    mesh = plsc.VectorSubcoreMesh(core_axis_name="c", subcore_axis_name="s")
    fm = pl.kernel(
        _fm_body,
        mesh=mesh,
        out_type=jax.ShapeDtypeStruct((B,), jnp.float32),
        scratch_types=[
            pltpu.VMEM((IDXW + 16,), jnp.int32),
            pltpu.VMEM((NF * VOCAB + 16,), jnp.float32),
            pltpu.VMEM((ROWS, D), jnp.float32),
            pltpu.VMEM((ROWS, D), jnp.float32),
            pltpu.VMEM((BPW,), jnp.float32),
            pltpu.SemaphoreType.DMA,
            pltpu.SemaphoreType.DMA,
        ],
    )
    raw = fm(cross_flat, lin_flat, flat_idx)
    return bias + raw


# speedup vs baseline: 1.0337x; 1.0337x over previous
import jax
import jax.numpy as jnp
from jax import lax
from jax.experimental import pallas as pl
from jax.experimental.pallas import tpu as pltpu
from jax.experimental.pallas import tpu_sc as plsc

NF = 26
VOCAB = 1000
B = 16384
D = 128
NW = 32
BPW = B // NW
G = 8
ROWS = G * NF
NGROUP = BPW // G
IDXP = BPW + 16


def _fm_body(cross_hbm, lin_hbm, *refs):
    feat_hbm = refs[:NF]
    out_hbm = refs[NF]
    idx_v, lin_v, buf_a, buf_b, out_v, sem_a, sem_b = refs[NF + 1:]

    wid = lax.axis_index("s") * 2 + lax.axis_index("c")

    for f in range(NF):
        pltpu.sync_copy(feat_hbm[f].at[pl.ds(wid * BPW, BPW)],
                        idx_v.at[pl.ds(f * IDXP, BPW)])
    pltpu.sync_copy(lin_hbm, lin_v.at[pl.ds(0, NF * VOCAB)])

    def issue(gg, buf, sem):
        for f in range(NF):
            pltpu.async_copy(
                cross_hbm.at[f].at[idx_v.at[pl.ds(f * IDXP + gg * G, G)]],
                buf.at[pl.ds(f * G, G)], sem)

    def drain(buf, sem):
        pltpu.make_async_copy(cross_hbm.at[0].at[pl.ds(0, ROWS)],
                              buf.at[pl.ds(0, ROWS)], sem).wait()

    lane = lax.iota(jnp.int32, 16)
    zero = jnp.zeros((16,), jnp.float32)

    def shuf(x, perm):
        return lax.gather(
            x, perm[:, None],
            lax.GatherDimensionNumbers(offset_dims=(),
                                       collapsed_slice_dims=(0,),
                                       start_index_map=(0,)),
            (1,), mode=lax.GatherScatterMode.PROMISE_IN_BOUNDS)

    def sample(buf, gg, s):
        def frow(r, carry):
            accs = list(carry)
            for k in range(8):
                x = buf[r * G + s, pl.ds(k * 16, 16)]
                accs[k] = accs[k] + x
                accs[8 + k] = accs[8 + k] + x * x
            return tuple(accs)

        accs = lax.fori_loop(0, NF, frow, (zero,) * 16, unroll=2)
        t = zero
        for k in range(8):
            t = t + (accs[k] * accs[k] - accs[8 + k])

        soff = gg * G + s
        la = zero
        lb = zero
        for f in range(0, NF, 2):
            ja = idx_v[pl.ds(f * IDXP + soff, 16)][0]
            jb = idx_v[pl.ds((f + 1) * IDXP + soff, 16)][0]
            la = la + lin_v[pl.ds(ja + f * VOCAB, 16)]
            lb = lb + lin_v[pl.ds(jb + (f + 1) * VOCAB, 16)]
        u = t + 2.0 * jnp.where(lane == 0, la + lb, 0.0)
        for sh in (8, 4, 2, 1):
            u = u + shuf(u, lane ^ sh)

        return 0.5 * u

    def process(buf, gg, r, lbase):
        def sbody(s, rr):
            rv = sample(buf, gg, s)
            return jnp.where(lane == lbase + s, rv, rr)
        return lax.fori_loop(0, G, sbody, r)

    issue(0, buf_a, sem_a)

    def gbody(g2, c):
        gg0 = g2 * 2
        issue(gg0 + 1, buf_b, sem_b)
        drain(buf_a, sem_a)
        r = process(buf_a, gg0, zero, 0)

        @pl.when(g2 < NGROUP // 2 - 1)
        def _():
            issue(gg0 + 2, buf_a, sem_a)

        drain(buf_b, sem_b)
        r = process(buf_b, gg0 + 1, r, G)
        out_v[pl.ds(g2 * 16, 16)] = r
        return c

    lax.fori_loop(0, NGROUP // 2, gbody, 0)

    pltpu.sync_copy(out_v, out_hbm.at[pl.ds(wid * BPW, BPW)])


def kernel(linear_tables, cross_tables, bias,
           feat_0, feat_1, feat_2, feat_3, feat_4, feat_5, feat_6,
           feat_7, feat_8, feat_9, feat_10, feat_11, feat_12, feat_13,
           feat_14, feat_15, feat_16, feat_17, feat_18, feat_19, feat_20,
           feat_21, feat_22, feat_23, feat_24, feat_25):
    feats = [feat_0, feat_1, feat_2, feat_3, feat_4, feat_5, feat_6,
             feat_7, feat_8, feat_9, feat_10, feat_11, feat_12, feat_13,
             feat_14, feat_15, feat_16, feat_17, feat_18, feat_19, feat_20,
             feat_21, feat_22, feat_23, feat_24, feat_25]
    lin_flat = linear_tables.reshape(NF * VOCAB)

    mesh = plsc.VectorSubcoreMesh(core_axis_name="c", subcore_axis_name="s")
    fm = pl.kernel(
        _fm_body,
        mesh=mesh,
        out_type=jax.ShapeDtypeStruct((B,), jnp.float32),
        scratch_types=[
            pltpu.VMEM((NF * IDXP,), jnp.int32),
            pltpu.VMEM((NF * VOCAB + 16,), jnp.float32),
            pltpu.VMEM((ROWS, D), jnp.float32),
            pltpu.VMEM((ROWS, D), jnp.float32),
            pltpu.VMEM((BPW,), jnp.float32),
            pltpu.SemaphoreType.DMA,
            pltpu.SemaphoreType.DMA,
        ],
    )
    raw = fm(cross_tables, lin_flat, *feats)
    return bias + raw


# speedup vs baseline: 1.1286x; 1.0918x over previous
import jax
import jax.numpy as jnp
from jax import lax
from jax.experimental import pallas as pl
from jax.experimental.pallas import tpu as pltpu
from jax.experimental.pallas import tpu_sc as plsc

NF = 26
VOCAB = 1000
B = 16384
D = 128
NW = 32
BPW = B // NW
G = 8
ROWS = G * NF
NGROUP = BPW // G
IDXP = BPW + 16


def _fm_body(cross_hbm, lin_hbm, *refs):
    feat_hbm = refs[:NF]
    out_hbm = refs[NF]
    idx_v, lin_v, buf_a, buf_b, out_v, sem_a, sem_b, sem_c = refs[NF + 1:]

    wid = lax.axis_index("s") * 2 + lax.axis_index("c")

    for f in range(NF):
        pltpu.async_copy(feat_hbm[f].at[pl.ds(wid * BPW, BPW)],
                         idx_v.at[pl.ds(f * IDXP, BPW)], sem_c)
    pltpu.async_copy(lin_hbm, lin_v.at[pl.ds(0, NF * VOCAB)], sem_c)
    for f in range(NF):
        pltpu.make_async_copy(feat_hbm[f].at[pl.ds(wid * BPW, BPW)],
                              idx_v.at[pl.ds(f * IDXP, BPW)], sem_c).wait()
    pltpu.make_async_copy(lin_hbm, lin_v.at[pl.ds(0, NF * VOCAB)],
                          sem_c).wait()

    def issue(gg, buf, sem):
        for f in range(NF):
            pltpu.async_copy(
                cross_hbm.at[f].at[idx_v.at[pl.ds(f * IDXP + gg * G, G)]],
                buf.at[pl.ds(f * G, G)], sem)

    def drain(buf, sem):
        pltpu.make_async_copy(cross_hbm.at[0].at[pl.ds(0, ROWS)],
                              buf.at[pl.ds(0, ROWS)], sem).wait()

    lane = lax.iota(jnp.int32, 16)
    zero = jnp.zeros((16,), jnp.float32)

    def shuf(x, perm):
        return lax.gather(
            x, perm[:, None],
            lax.GatherDimensionNumbers(offset_dims=(),
                                       collapsed_slice_dims=(0,),
                                       start_index_map=(0,)),
            (1,), mode=lax.GatherScatterMode.PROMISE_IN_BOUNDS)

    def sample(buf, gg, s):
        def frow(r, carry):
            accs = list(carry)
            for k in range(8):
                x = buf[r * G + s, pl.ds(k * 16, 16)]
                accs[k] = accs[k] + x
                accs[8 + k] = accs[8 + k] + x * x
            return tuple(accs)

        accs = lax.fori_loop(0, NF, frow, (zero,) * 16, unroll=2)
        t = zero
        for k in range(8):
            t = t + (accs[k] * accs[k] - accs[8 + k])

        soff = gg * G + s
        la = zero
        lb = zero
        for f in range(0, NF, 2):
            ja = idx_v[pl.ds(f * IDXP + soff, 16)][0]
            jb = idx_v[pl.ds((f + 1) * IDXP + soff, 16)][0]
            la = la + lin_v[pl.ds(ja + f * VOCAB, 16)]
            lb = lb + lin_v[pl.ds(jb + (f + 1) * VOCAB, 16)]
        u = t + 2.0 * jnp.where(lane == 0, la + lb, 0.0)
        for sh in (8, 4, 2, 1):
            u = u + shuf(u, lane ^ sh)

        return 0.5 * u

    def process(buf, gg, r, lbase):
        def sbody(s, rr):
            rv = sample(buf, gg, s)
            return jnp.where(lane == lbase + s, rv, rr)
        return lax.fori_loop(0, G, sbody, r)

    issue(0, buf_a, sem_a)

    def gbody(g2, c):
        gg0 = g2 * 2
        issue(gg0 + 1, buf_b, sem_b)
        drain(buf_a, sem_a)
        r = process(buf_a, gg0, zero, 0)

        @pl.when(g2 < NGROUP // 2 - 1)
        def _():
            issue(gg0 + 2, buf_a, sem_a)

        drain(buf_b, sem_b)
        r = process(buf_b, gg0 + 1, r, G)
        out_v[pl.ds(g2 * 16, 16)] = r
        return c

    lax.fori_loop(0, NGROUP // 2, gbody, 0)

    pltpu.sync_copy(out_v, out_hbm.at[pl.ds(wid * BPW, BPW)])


def kernel(linear_tables, cross_tables, bias,
           feat_0, feat_1, feat_2, feat_3, feat_4, feat_5, feat_6,
           feat_7, feat_8, feat_9, feat_10, feat_11, feat_12, feat_13,
           feat_14, feat_15, feat_16, feat_17, feat_18, feat_19, feat_20,
           feat_21, feat_22, feat_23, feat_24, feat_25):
    feats = [feat_0, feat_1, feat_2, feat_3, feat_4, feat_5, feat_6,
             feat_7, feat_8, feat_9, feat_10, feat_11, feat_12, feat_13,
             feat_14, feat_15, feat_16, feat_17, feat_18, feat_19, feat_20,
             feat_21, feat_22, feat_23, feat_24, feat_25]
    lin_flat = linear_tables.reshape(NF * VOCAB)

    mesh = plsc.VectorSubcoreMesh(core_axis_name="c", subcore_axis_name="s")
    fm = pl.kernel(
        _fm_body,
        mesh=mesh,
        out_type=jax.ShapeDtypeStruct((B,), jnp.float32),
        scratch_types=[
            pltpu.VMEM((NF * IDXP,), jnp.int32),
            pltpu.VMEM((NF * VOCAB + 16,), jnp.float32),
            pltpu.VMEM((ROWS, D), jnp.float32),
            pltpu.VMEM((ROWS, D), jnp.float32),
            pltpu.VMEM((BPW,), jnp.float32),
            pltpu.SemaphoreType.DMA,
            pltpu.SemaphoreType.DMA,
            pltpu.SemaphoreType.DMA,
        ],
    )
    raw = fm(cross_tables, lin_flat, *feats)
    return bias + raw


# speedup vs baseline: 1.1426x; 1.0124x over previous
import jax
import jax.numpy as jnp
from jax import lax
from jax.experimental import pallas as pl
from jax.experimental.pallas import tpu as pltpu
from jax.experimental.pallas import tpu_sc as plsc

NF = 26
VOCAB = 1000
B = 16384
D = 128
NW = 32
BPW = B // NW
G = 8
ROWS = G * NF
NGROUP = BPW // G
IDXP = BPW + 16


def _fm_body(cross_hbm, lin_hbm, *refs):
    feat_hbm = refs[:NF]
    out_hbm = refs[NF]
    idx_v, lin_v, buf_a, buf_b, out_v, sem_a, sem_b, sem_c = refs[NF + 1:]

    wid = lax.axis_index("s") * 2 + lax.axis_index("c")

    for f in range(NF):
        pltpu.async_copy(feat_hbm[f].at[pl.ds(wid * BPW, BPW)],
                         idx_v.at[pl.ds(f * IDXP, BPW)], sem_c)
    pltpu.async_copy(lin_hbm, lin_v.at[pl.ds(0, NF * VOCAB)], sem_c)
    for f in range(NF):
        pltpu.make_async_copy(feat_hbm[f].at[pl.ds(wid * BPW, BPW)],
                              idx_v.at[pl.ds(f * IDXP, BPW)], sem_c).wait()

    def issue(gg, buf, sem):
        for f in range(NF):
            pltpu.async_copy(
                cross_hbm.at[f].at[idx_v.at[pl.ds(f * IDXP + gg * G, G)]],
                buf.at[pl.ds(f * G, G)], sem)

    def drain(buf, sem):
        pltpu.make_async_copy(cross_hbm.at[0].at[pl.ds(0, ROWS)],
                              buf.at[pl.ds(0, ROWS)], sem).wait()

    lane = lax.iota(jnp.int32, 16)
    zero = jnp.zeros((16,), jnp.float32)

    def shuf(x, perm):
        return lax.gather(
            x, perm[:, None],
            lax.GatherDimensionNumbers(offset_dims=(),
                                       collapsed_slice_dims=(0,),
                                       start_index_map=(0,)),
            (1,), mode=lax.GatherScatterMode.PROMISE_IN_BOUNDS)

    def sample(buf, gg, s):
        def frow(r, carry):
            accs = list(carry)
            for k in range(8):
                x = buf[r * G + s, pl.ds(k * 16, 16)]
                accs[k] = accs[k] + x
                accs[8 + k] = accs[8 + k] + x * x
            return tuple(accs)

        accs = lax.fori_loop(0, NF, frow, (zero,) * 16, unroll=2)
        t = zero
        for k in range(8):
            t = t + (accs[k] * accs[k] - accs[8 + k])

        soff = gg * G + s
        la = zero
        lb = zero
        for f in range(0, NF, 2):
            ja = idx_v[pl.ds(f * IDXP + soff, 16)][0]
            jb = idx_v[pl.ds((f + 1) * IDXP + soff, 16)][0]
            la = la + lin_v[pl.ds(ja + f * VOCAB, 16)]
            lb = lb + lin_v[pl.ds(jb + (f + 1) * VOCAB, 16)]
        u = t + 2.0 * jnp.where(lane == 0, la + lb, 0.0)
        for sh in (8, 4, 2, 1):
            u = u + shuf(u, lane ^ sh)

        return 0.5 * u

    def process(buf, gg, r, lbase):
        def sbody(s, rr):
            rv = sample(buf, gg, s)
            return jnp.where(lane == lbase + s, rv, rr)
        return lax.fori_loop(0, G, sbody, r)

    issue(0, buf_a, sem_a)
    pltpu.make_async_copy(lin_hbm, lin_v.at[pl.ds(0, NF * VOCAB)],
                          sem_c).wait()

    def gbody(g2, c):
        gg0 = g2 * 2
        issue(gg0 + 1, buf_b, sem_b)
        drain(buf_a, sem_a)
        r = process(buf_a, gg0, zero, 0)

        @pl.when(g2 < NGROUP // 2 - 1)
        def _():
            issue(gg0 + 2, buf_a, sem_a)

        drain(buf_b, sem_b)
        r = process(buf_b, gg0 + 1, r, G)
        out_v[pl.ds(g2 * 16, 16)] = r
        return c

    lax.fori_loop(0, NGROUP // 2, gbody, 0)

    pltpu.sync_copy(out_v, out_hbm.at[pl.ds(wid * BPW, BPW)])


def kernel(linear_tables, cross_tables, bias,
           feat_0, feat_1, feat_2, feat_3, feat_4, feat_5, feat_6,
           feat_7, feat_8, feat_9, feat_10, feat_11, feat_12, feat_13,
           feat_14, feat_15, feat_16, feat_17, feat_18, feat_19, feat_20,
           feat_21, feat_22, feat_23, feat_24, feat_25):
    feats = [feat_0, feat_1, feat_2, feat_3, feat_4, feat_5, feat_6,
             feat_7, feat_8, feat_9, feat_10, feat_11, feat_12, feat_13,
             feat_14, feat_15, feat_16, feat_17, feat_18, feat_19, feat_20,
             feat_21, feat_22, feat_23, feat_24, feat_25]
    lin_flat = linear_tables.reshape(NF * VOCAB)

    mesh = plsc.VectorSubcoreMesh(core_axis_name="c", subcore_axis_name="s")
    fm = pl.kernel(
        _fm_body,
        mesh=mesh,
        out_type=jax.ShapeDtypeStruct((B,), jnp.float32),
        scratch_types=[
            pltpu.VMEM((NF * IDXP,), jnp.int32),
            pltpu.VMEM((NF * VOCAB + 16,), jnp.float32),
            pltpu.VMEM((ROWS, D), jnp.float32),
            pltpu.VMEM((ROWS, D), jnp.float32),
            pltpu.VMEM((BPW,), jnp.float32),
            pltpu.SemaphoreType.DMA,
            pltpu.SemaphoreType.DMA,
            pltpu.SemaphoreType.DMA,
        ],
    )
    raw = fm(cross_tables, lin_flat, *feats)
    return bias + raw
